# Initial kernel scaffold; baseline (speedup 1.0000x reference)
import jax
import jax.numpy as jnp
from jax.experimental import pallas as pl


def kernel(x, edge_index, edges, edges_neg, W1, b1, W2, b2, B1, bb1, B2, bb2, Wc, bc):
    raise NotImplementedError("write your pallas kernel here")


# speedup vs baseline: 7.0743x; 7.0743x over previous
import functools

import jax
import jax.numpy as jnp
from jax import lax
from jax.experimental import pallas as pl
from jax.experimental.pallas import tpu as pltpu
from jax.experimental.pallas import tpu_sc as plsc

N = 10000
E = 320000
V = 3
D = 128
P = 50000
ALPHA = 0.5

NC = 2
NS = 16
L = 16
NW = NC * NS

NP = 10240
PAD = N
K = 128
EPT = E // NW
NCH = 80
HSZ = V * NP
ZR = 64
IB = 16

TP = 2 * V * P
NCHL = 74
PPT = NCHL * K
TPP = NW * PPT

BN = 1024
GRID = NP // BN
BF = 12500


DZR = 32


def _deg_body(dst_hbm, hist_hbm, didx_v, buf_v, zbuf_v, h16_v, acc_sh):
    c = lax.axis_index("c")
    s = lax.axis_index("s")
    zero = jnp.zeros((L,), jnp.float32)
    one = jnp.ones((L,), jnp.float32)

    @pl.loop(0, DZR)
    def _zfill(r):
        for t in range(D // L):
            zbuf_v[r, pl.ds(t * L, L)] = zero

    for v in range(V):
        plsc.subcore_barrier()
        for q in range(640 // DZR):
            pltpu.sync_copy(zbuf_v, acc_sh.at[pl.ds(s * 640 + q * DZR, DZR)])

        @pl.loop(0, K)
        def _ofill(r):
            for t in range(D // L):
                buf_v[r, pl.ds(t * L, L)] = one

        plsc.subcore_barrier()

        @pl.loop(0, NCH // IB)
        def _stage(st):
            pltpu.sync_copy(dst_hbm.at[v, c, s, pl.ds(st * IB, IB)], didx_v)

            @pl.loop(0, IB)
            def _chunks(j):
                pltpu.sync_copy(buf_v, acc_sh.at[didx_v.at[j]], add=True)

        plsc.subcore_barrier()
        for q in range(640 // K):
            pltpu.sync_copy(acc_sh.at[pl.ds(s * 640 + q * K, K)], buf_v)

            @pl.loop(0, K)
            def _cp(r):
                h16_v[r, :] = buf_v[r, pl.ds(0, L)]

            pltpu.sync_copy(h16_v,
                            hist_hbm.at[c, v, pl.ds(s * 640 + q * K, K)])


@functools.cache
def _mesh():
    return plsc.VectorSubcoreMesh(
        core_axis_name="c", subcore_axis_name="s",
        num_cores=NC, num_subcores=NS)


@functools.cache
def _deg_built():
    return pl.kernel(
        _deg_body,
        out_type=jax.ShapeDtypeStruct((NC, V, NP, L), jnp.float32),
        mesh=_mesh(),
        scratch_types=[
            pltpu.VMEM((IB, K), jnp.int32),
            pltpu.VMEM((K, D), jnp.float32),
            pltpu.VMEM((DZR, D), jnp.float32),
            pltpu.VMEM((K, L), jnp.float32),
            pltpu.VMEM_SHARED((NP, D), jnp.float32),
        ])


def _deg_kernel(dst_l):
    return _deg_built()(dst_l)


def _scat_body(y_hbm, src_hbm, dst_hbm, out_hbm,
               sidx_v, didx_v, rows0_v, rows1_v, zrow_v, acc_sh, sem0, sem1):
    c = lax.axis_index("c")
    s = lax.axis_index("s")
    zero = jnp.zeros((L,), jnp.float32)

    @pl.loop(0, ZR)
    def _fill(r):
        for t in range(D // L):
            zrow_v[r, pl.ds(t * L, L)] = zero

    for v in range(V):
        plsc.subcore_barrier()
        for q in range(640 // ZR):
            pltpu.sync_copy(zrow_v, acc_sh.at[pl.ds(s * 640 + q * ZR, ZR)])
        plsc.subcore_barrier()

        @pl.loop(0, NCH // IB)
        def _stage(st):
            pltpu.sync_copy(src_hbm.at[v, c, s, pl.ds(st * IB, IB)], sidx_v)
            pltpu.sync_copy(dst_hbm.at[v, c, s, pl.ds(st * IB, IB)], didx_v)

            @pl.loop(0, IB, step=2)
            def _chunks(j):
                c0 = pltpu.async_copy(y_hbm.at[sidx_v.at[j]], rows0_v, sem0)
                c1 = pltpu.async_copy(y_hbm.at[sidx_v.at[j + 1]], rows1_v, sem1)
                c0.wait()
                pltpu.sync_copy(rows0_v, acc_sh.at[didx_v.at[j]], add=True)
                c1.wait()
                pltpu.sync_copy(rows1_v, acc_sh.at[didx_v.at[j + 1]], add=True)

        plsc.subcore_barrier()
        for q in range(640 // K):
            pltpu.sync_copy(acc_sh.at[pl.ds(s * 640 + q * K, K)], rows0_v)
            pltpu.sync_copy(rows0_v, out_hbm.at[c, v, pl.ds(s * 640 + q * K, K)])


@functools.cache
def _scat_built():
    return pl.kernel(
        _scat_body,
        out_type=jax.ShapeDtypeStruct((NC, V, NP, D), jnp.float32),
        mesh=_mesh(),
        scratch_types=[
            pltpu.VMEM((IB, K), jnp.int32),
            pltpu.VMEM((IB, K), jnp.int32),
            pltpu.VMEM((K, D), jnp.float32),
            pltpu.VMEM((K, D), jnp.float32),
            pltpu.VMEM((ZR, D), jnp.float32),
            pltpu.VMEM_SHARED((NP, D), jnp.float32),
            pltpu.SemaphoreType.DMA,
            pltpu.SemaphoreType.DMA,
        ])


def _scat_kernel(y_flat, src_g, dst_l):
    return _scat_built()(y_flat, src_g, dst_l)


def _lp_body(xf_hbm, aidx_hbm, bidx_hbm, out_hbm,
             aidx_v, bidx_v, ra_v, rb_v, sc_v, sa, sb):
    c = lax.axis_index("c")
    s = lax.axis_index("s")
    pltpu.sync_copy(aidx_hbm.at[c, s], aidx_v)
    pltpu.sync_copy(bidx_hbm.at[c, s], bidx_v)

    @pl.loop(0, NCHL)
    def _chunks(j):
        ca = pltpu.async_copy(xf_hbm.at[aidx_v.at[j]], ra_v, sa)
        cb = pltpu.async_copy(xf_hbm.at[bidx_v.at[j]], rb_v, sb)
        ca.wait()
        cb.wait()

        @pl.loop(0, K)
        def _pairs(p):
            acc = ra_v[p, pl.ds(0, L)] * rb_v[p, pl.ds(0, L)]
            for t in range(1, D // L):
                acc = acc + ra_v[p, pl.ds(t * L, L)] * rb_v[p, pl.ds(t * L, L)]
            sc_v[p, :] = acc

        pltpu.sync_copy(sc_v, out_hbm.at[c, s, j])


@functools.cache
def _lp_built():
    return pl.kernel(
        _lp_body,
        out_type=jax.ShapeDtypeStruct((NC, NS, NCHL, K, L), jnp.float32),
        mesh=_mesh(),
        scratch_types=[
            pltpu.VMEM((NCHL, K), jnp.int32),
            pltpu.VMEM((NCHL, K), jnp.int32),
            pltpu.VMEM((K, D), jnp.float32),
            pltpu.VMEM((K, D), jnp.float32),
            pltpu.VMEM((K, L), jnp.float32),
            pltpu.SemaphoreType.DMA,
            pltpu.SemaphoreType.DMA,
        ])


def _lp_kernel(xf_flat, a_pad, b_pad):
    return _lp_built()(xf_flat, a_pad, b_pad)


def _tc_d_body(p_ref, o_ref):
    o_ref[...] = jnp.sum(p_ref[...], axis=1)


def _elu(x):
    return jnp.where(x > 0, x, jnp.exp(x) - 1.0)


def _mm_t(a, w):
    return lax.dot_general(a, w, (((1,), (1,)), ((), ())),
                           preferred_element_type=jnp.float32)


def _att_rows(w_ref, bmat_ref, bb_ref):
    m_rows = []
    for i in range(V):
        t_i = lax.dot_general(w_ref[i], bmat_ref[...], (((1,), (0,)), ((), ())),
                              preferred_element_type=jnp.float32)
        for j in range(V):
            tot = jnp.sum(t_i * w_ref[j], axis=0, keepdims=True)
            tot = jnp.sum(tot, axis=1, keepdims=True)
            m_rows.append(jnp.broadcast_to(tot, (1, D)))
    rows = []
    for i in range(V):
        r = [m_rows[3 * i + j] + jnp.float32(D) * bb_ref[...] for j in range(V)]
        mx = jnp.maximum(jnp.maximum(r[0], r[1]), r[2])
        e = [jnp.exp(rj - mx) for rj in r]
        tot = e[0] + e[1] + e[2]
        rows.extend([ej / tot for ej in e])
    return jnp.concatenate(rows, axis=0)


def _tc_a_body(x_ref, hist_ref, w1_ref, b1m_ref, bb1_ref,
               xw1_ref, y1_ref, dis_ref, dinv_ref, att1_ref):
    g = pl.program_id(0)
    deg = jnp.sum(hist_ref[...], axis=(0, 3)) * (1.0 / L) + 1.0
    dis = lax.rsqrt(deg)
    dinv = 1.0 / deg
    dis_ref[...] = dis
    dinv_ref[...] = dinv
    xb = x_ref[...]
    for v in range(V):
        xw = _mm_t(xb, w1_ref[v])
        xw1_ref[v] = xw
        y1_ref[v] = xw * dis[v][:, None]

    @pl.when(g == 0)
    def _att():
        att1_ref[...] = _att_rows(w1_ref, b1m_ref, bb1_ref)


def _tc_b_body(part_ref, xw1_ref, dis_ref, dinv_ref, b1_ref, att_ref,
               wc_ref, bc_ref, w2_ref, b2m_ref, bb2_ref,
               xw2_ref, y2_ref, att2_ref):
    g = pl.program_id(0)
    dis = dis_ref[...]
    dinv = dinv_ref[...]
    xms = []
    for v in range(V):
        accv = part_ref[0, v] + part_ref[1, v]
        out1 = (accv * dis[v][:, None] + xw1_ref[v] * dinv[v][:, None]
                + b1_ref[v][None, :])
        temp = (att_ref[3 * v][None, :] * xw1_ref[0]
                + att_ref[3 * v + 1][None, :] * xw1_ref[1]
                + att_ref[3 * v + 2][None, :] * xw1_ref[2])
        xms.append((1.0 - ALPHA) + _elu(out1) + _elu(ALPHA * temp))
    xcat = jnp.concatenate(xms, axis=1)
    xc = _elu(_mm_t(xcat, wc_ref[...]) + bc_ref[...])
    for v in range(V):
        xw2 = _mm_t(xc, w2_ref[v])
        xw2_ref[v] = xw2
        y2_ref[v] = xw2 * dis[v][:, None]

    @pl.when(g == 0)
    def _att():
        att2_ref[...] = _att_rows(w2_ref, b2m_ref, bb2_ref)


def _tc_c_body(part_ref, xw2_ref, dis_ref, dinv_ref, b2_ref, att_ref, xf_ref):
    dis = dis_ref[...]
    dinv = dinv_ref[...]
    for v in range(V):
        accv = part_ref[0, v] + part_ref[1, v]
        out2 = (accv * dis[v][:, None] + xw2_ref[v] * dinv[v][:, None]
                + b2_ref[v][None, :])
        temp = (att_ref[3 * v][None, :] * xw2_ref[0]
                + att_ref[3 * v + 1][None, :] * xw2_ref[1]
                + att_ref[3 * v + 2][None, :] * xw2_ref[2])
        xf_ref[v] = (1.0 - ALPHA) + out2 + _elu(ALPHA * temp)


def _edges_pad(a, shift):
    a = a.reshape(V, NW, EPT)
    a = jnp.pad(a, ((0, 0), (0, 0), (0, NCH * K - EPT)), constant_values=PAD)
    if shift:
        a = a + (jnp.arange(V, dtype=jnp.int32) * NP)[:, None, None]
    return a.reshape(V, NC, NS, NCH, K)


def kernel(x, edge_index, edges, edges_neg, W1, b1, W2, b2,
           B1, bb1, B2, bb2, Wc, bc):
    f32 = jnp.float32
    x_p = jnp.pad(x, ((0, NP - N), (0, 0)))
    src_g = _edges_pad(edge_index[:, 0, :].astype(jnp.int32), True)
    dst_l = _edges_pad(edge_index[:, 1, :].astype(jnp.int32), False)
    bb1_row = jnp.broadcast_to(bb1.astype(f32).reshape(1, 1), (1, D))
    bb2_row = jnp.broadcast_to(bb2.astype(f32).reshape(1, 1), (1, D))

    hist = _deg_kernel(dst_l)

    full = lambda *dims: pl.BlockSpec(dims, lambda g: tuple(0 for _ in dims))
    xw1, y1, dis, dinv, att1 = pl.pallas_call(
        _tc_a_body,
        grid=(GRID,),
        in_specs=[
            pl.BlockSpec((BN, D), lambda g: (g, 0)),
            pl.BlockSpec((NC, V, BN, L), lambda g: (0, 0, g, 0)),
            full(V, D, D),
            full(D, D),
            full(1, D),
        ],
        out_specs=[
            pl.BlockSpec((V, BN, D), lambda g: (0, g, 0)),
            pl.BlockSpec((V, BN, D), lambda g: (0, g, 0)),
            pl.BlockSpec((V, BN), lambda g: (0, g)),
            pl.BlockSpec((V, BN), lambda g: (0, g)),
            full(9, D),
        ],
        out_shape=[
            jax.ShapeDtypeStruct((V, NP, D), f32),
            jax.ShapeDtypeStruct((V, NP, D), f32),
            jax.ShapeDtypeStruct((V, NP), f32),
            jax.ShapeDtypeStruct((V, NP), f32),
            jax.ShapeDtypeStruct((9, D), f32),
        ],
    )(x_p, hist, W1, B1, bb1_row)

    part1 = _scat_kernel(y1.reshape(V * NP, D), src_g, dst_l)

    xw2, y2, att2 = pl.pallas_call(
        _tc_b_body,
        grid=(GRID,),
        in_specs=[
            pl.BlockSpec((NC, V, BN, D), lambda g: (0, 0, g, 0)),
            pl.BlockSpec((V, BN, D), lambda g: (0, g, 0)),
            pl.BlockSpec((V, BN), lambda g: (0, g)),
            pl.BlockSpec((V, BN), lambda g: (0, g)),
            full(V, D),
            full(9, D),
            full(D, V * D),
            full(1, D),
            full(V, D, D),
            full(D, D),
            full(1, D),
        ],
        out_specs=[
            pl.BlockSpec((V, BN, D), lambda g: (0, g, 0)),
            pl.BlockSpec((V, BN, D), lambda g: (0, g, 0)),
            full(9, D),
        ],
        out_shape=[
            jax.ShapeDtypeStruct((V, NP, D), f32),
            jax.ShapeDtypeStruct((V, NP, D), f32),
            jax.ShapeDtypeStruct((9, D), f32),
        ],
    )(part1, xw1, dis, dinv, b1, att1, Wc, bc.reshape(1, D), W2, B2, bb2_row)

    part2 = _scat_kernel(y2.reshape(V * NP, D), src_g, dst_l)

    xf = pl.pallas_call(
        _tc_c_body,
        grid=(GRID,),
        in_specs=[
            pl.BlockSpec((NC, V, BN, D), lambda g: (0, 0, g, 0)),
            pl.BlockSpec((V, BN, D), lambda g: (0, g, 0)),
            pl.BlockSpec((V, BN), lambda g: (0, g)),
            pl.BlockSpec((V, BN), lambda g: (0, g)),
            full(V, D),
            full(9, D),
        ],
        out_specs=pl.BlockSpec((V, BN, D), lambda g: (0, g, 0)),
        out_shape=jax.ShapeDtypeStruct((V, NP, D), f32),
    )(part2, xw2, dis, dinv, b2, att2)

    shiftv = (jnp.arange(V, dtype=jnp.int32) * NP)[:, None]
    a_flat = (jnp.concatenate([edges[:, :, 0], edges_neg[:, :, 0]], axis=1)
              .astype(jnp.int32) + shiftv).reshape(TP)
    b_flat = (jnp.concatenate([edges[:, :, 1], edges_neg[:, :, 1]], axis=1)
              .astype(jnp.int32) + shiftv).reshape(TP)
    a_pad = jnp.pad(a_flat, (0, TPP - TP),
                    constant_values=PAD).reshape(NC, NS, NCHL, K)
    b_pad = jnp.pad(b_flat, (0, TPP - TP),
                    constant_values=PAD).reshape(NC, NS, NCHL, K)

    part_lp = _lp_kernel(xf.reshape(V * NP, D), a_pad, b_pad)

    BM = 8192
    flat = pl.pallas_call(
        _tc_d_body,
        grid=(TPP // BM,),
        in_specs=[pl.BlockSpec((BM, L), lambda i: (i, 0))],
        out_specs=pl.BlockSpec((BM,), lambda i: (i,)),
        out_shape=jax.ShapeDtypeStruct((TPP,), f32),
    )(part_lp.reshape(TPP, L))
    return flat[:TP].reshape(V, 2 * P)


# speedup vs baseline: 7.3916x; 1.0449x over previous
import functools

import jax
import jax.numpy as jnp
from jax import lax
from jax.experimental import pallas as pl
from jax.experimental.pallas import tpu as pltpu
from jax.experimental.pallas import tpu_sc as plsc

N = 10000
E = 320000
V = 3
D = 128
P = 50000
ALPHA = 0.5

NC = 2
NS = 16
L = 16
NW = NC * NS

NP = 10240
PAD = N
K = 128
EPT = E // NW
NCH = 80
HSZ = V * NP
ZR = 64
IB = 16

TP = 2 * V * P
NCHL = 74
PPT = NCHL * K
TPP = NW * PPT

BN = 1024
GRID = NP // BN
BF = 12500


DZR = 32


def _deg_body(dst_hbm, hist_hbm, didx_v, buf_v, zbuf_v, h16_v, acc_sh):
    c = lax.axis_index("c")
    s = lax.axis_index("s")
    zero = jnp.zeros((L,), jnp.float32)
    one = jnp.ones((L,), jnp.float32)

    @pl.loop(0, DZR)
    def _zfill(r):
        for t in range(D // L):
            zbuf_v[r, pl.ds(t * L, L)] = zero

    for v in range(V):
        plsc.subcore_barrier()
        for q in range(640 // DZR):
            pltpu.sync_copy(zbuf_v, acc_sh.at[pl.ds(s * 640 + q * DZR, DZR)])

        @pl.loop(0, K)
        def _ofill(r):
            for t in range(D // L):
                buf_v[r, pl.ds(t * L, L)] = one

        plsc.subcore_barrier()

        @pl.loop(0, NCH // IB)
        def _stage(st):
            pltpu.sync_copy(dst_hbm.at[v, c, s, pl.ds(st * IB, IB)], didx_v)

            @pl.loop(0, IB)
            def _chunks(j):
                pltpu.sync_copy(buf_v, acc_sh.at[didx_v.at[j]], add=True)

        plsc.subcore_barrier()
        for q in range(640 // K):
            pltpu.sync_copy(acc_sh.at[pl.ds(s * 640 + q * K, K)], buf_v)

            @pl.loop(0, K)
            def _cp(r):
                h16_v[r, :] = buf_v[r, pl.ds(0, L)]

            pltpu.sync_copy(h16_v,
                            hist_hbm.at[c, v, pl.ds(s * 640 + q * K, K)])


@functools.cache
def _mesh():
    return plsc.VectorSubcoreMesh(
        core_axis_name="c", subcore_axis_name="s",
        num_cores=NC, num_subcores=NS)


@functools.cache
def _deg_built():
    return pl.kernel(
        _deg_body,
        out_type=jax.ShapeDtypeStruct((NC, V, NP, L), jnp.float32),
        mesh=_mesh(),
        scratch_types=[
            pltpu.VMEM((IB, K), jnp.int32),
            pltpu.VMEM((K, D), jnp.float32),
            pltpu.VMEM((DZR, D), jnp.float32),
            pltpu.VMEM((K, L), jnp.float32),
            pltpu.VMEM_SHARED((NP, D), jnp.float32),
        ])


def _deg_kernel(dst_l):
    return _deg_built()(dst_l)


def _scat_body(y_hbm, src_hbm, dst_hbm, out_hbm,
               sidx_v, didx_v, rows0_v, rows1_v, zrow_v, acc_sh,
               sem0, sem1, sem2, sem3):
    c = lax.axis_index("c")
    s = lax.axis_index("s")
    zero = jnp.zeros((L,), jnp.float32)

    @pl.loop(0, ZR)
    def _fill(r):
        for t in range(D // L):
            zrow_v[r, pl.ds(t * L, L)] = zero

    for v in range(V):
        plsc.subcore_barrier()
        for q in range(640 // ZR):
            pltpu.sync_copy(zrow_v, acc_sh.at[pl.ds(s * 640 + q * ZR, ZR)])
        plsc.subcore_barrier()

        @pl.loop(0, NCH // IB)
        def _stage(st):
            pltpu.sync_copy(src_hbm.at[v, c, s, pl.ds(st * IB, IB)], sidx_v)
            pltpu.sync_copy(dst_hbm.at[v, c, s, pl.ds(st * IB, IB)], didx_v)
            pltpu.async_copy(y_hbm.at[sidx_v.at[0]], rows0_v, sem0)
            pltpu.async_copy(y_hbm.at[sidx_v.at[1]], rows1_v, sem1)

            @pl.loop(0, (IB - 2) // 2)
            def _chunks(h):
                j = h * 2
                pltpu.make_async_copy(y_hbm.at[sidx_v.at[j]], rows0_v, sem0).wait()
                pltpu.async_copy(rows0_v, acc_sh.at[didx_v.at[j]], sem2, add=True)
                pltpu.make_async_copy(y_hbm.at[sidx_v.at[j + 1]], rows1_v, sem1).wait()
                pltpu.async_copy(rows1_v, acc_sh.at[didx_v.at[j + 1]], sem3, add=True)
                pltpu.make_async_copy(rows0_v, acc_sh.at[didx_v.at[j]], sem2).wait()
                pltpu.async_copy(y_hbm.at[sidx_v.at[j + 2]], rows0_v, sem0)
                pltpu.make_async_copy(rows1_v, acc_sh.at[didx_v.at[j + 1]], sem3).wait()
                pltpu.async_copy(y_hbm.at[sidx_v.at[j + 3]], rows1_v, sem1)

            je = IB - 2
            pltpu.make_async_copy(y_hbm.at[sidx_v.at[je]], rows0_v, sem0).wait()
            pltpu.async_copy(rows0_v, acc_sh.at[didx_v.at[je]], sem2, add=True)
            pltpu.make_async_copy(y_hbm.at[sidx_v.at[je + 1]], rows1_v, sem1).wait()
            pltpu.async_copy(rows1_v, acc_sh.at[didx_v.at[je + 1]], sem3, add=True)
            pltpu.make_async_copy(rows0_v, acc_sh.at[didx_v.at[je]], sem2).wait()
            pltpu.make_async_copy(rows1_v, acc_sh.at[didx_v.at[je + 1]], sem3).wait()

        plsc.subcore_barrier()
        for q in range(640 // K):
            pltpu.sync_copy(acc_sh.at[pl.ds(s * 640 + q * K, K)], rows0_v)
            pltpu.sync_copy(rows0_v, out_hbm.at[c, v, pl.ds(s * 640 + q * K, K)])


@functools.cache
def _scat_built():
    return pl.kernel(
        _scat_body,
        out_type=jax.ShapeDtypeStruct((NC, V, NP, D), jnp.float32),
        mesh=_mesh(),
        scratch_types=[
            pltpu.VMEM((IB, K), jnp.int32),
            pltpu.VMEM((IB, K), jnp.int32),
            pltpu.VMEM((K, D), jnp.float32),
            pltpu.VMEM((K, D), jnp.float32),
            pltpu.VMEM((ZR, D), jnp.float32),
            pltpu.VMEM_SHARED((NP, D), jnp.float32),
            pltpu.SemaphoreType.DMA,
            pltpu.SemaphoreType.DMA,
            pltpu.SemaphoreType.DMA,
            pltpu.SemaphoreType.DMA,
        ])


def _scat_kernel(y_flat, src_g, dst_l):
    return _scat_built()(y_flat, src_g, dst_l)


def _lp_compute(ra_v, rb_v, sc_v):
    @pl.loop(0, K)
    def _pairs(p):
        acc = ra_v[p, pl.ds(0, L)] * rb_v[p, pl.ds(0, L)]
        for t in range(1, D // L):
            acc = acc + ra_v[p, pl.ds(t * L, L)] * rb_v[p, pl.ds(t * L, L)]
        sc_v[p, :] = acc


def _lp_body(xf_hbm, aidx_hbm, bidx_hbm, out_hbm,
             aidx_v, bidx_v, ra0_v, rb0_v, ra1_v, rb1_v, sc_v,
             sa0, sb0, sa1, sb1):
    c = lax.axis_index("c")
    s = lax.axis_index("s")
    pltpu.sync_copy(aidx_hbm.at[c, s], aidx_v)
    pltpu.sync_copy(bidx_hbm.at[c, s], bidx_v)

    pltpu.async_copy(xf_hbm.at[aidx_v.at[0]], ra0_v, sa0)
    pltpu.async_copy(xf_hbm.at[bidx_v.at[0]], rb0_v, sb0)
    pltpu.async_copy(xf_hbm.at[aidx_v.at[1]], ra1_v, sa1)
    pltpu.async_copy(xf_hbm.at[bidx_v.at[1]], rb1_v, sb1)

    @pl.loop(0, (NCHL - 2) // 2)
    def _chunks(h):
        j = h * 2
        pltpu.make_async_copy(xf_hbm.at[aidx_v.at[j]], ra0_v, sa0).wait()
        pltpu.make_async_copy(xf_hbm.at[bidx_v.at[j]], rb0_v, sb0).wait()
        _lp_compute(ra0_v, rb0_v, sc_v)
        pltpu.sync_copy(sc_v, out_hbm.at[c, s, j])
        pltpu.async_copy(xf_hbm.at[aidx_v.at[j + 2]], ra0_v, sa0)
        pltpu.async_copy(xf_hbm.at[bidx_v.at[j + 2]], rb0_v, sb0)
        pltpu.make_async_copy(xf_hbm.at[aidx_v.at[j + 1]], ra1_v, sa1).wait()
        pltpu.make_async_copy(xf_hbm.at[bidx_v.at[j + 1]], rb1_v, sb1).wait()
        _lp_compute(ra1_v, rb1_v, sc_v)
        pltpu.sync_copy(sc_v, out_hbm.at[c, s, j + 1])
        pltpu.async_copy(xf_hbm.at[aidx_v.at[j + 3]], ra1_v, sa1)
        pltpu.async_copy(xf_hbm.at[bidx_v.at[j + 3]], rb1_v, sb1)

    je = NCHL - 2
    pltpu.make_async_copy(xf_hbm.at[aidx_v.at[je]], ra0_v, sa0).wait()
    pltpu.make_async_copy(xf_hbm.at[bidx_v.at[je]], rb0_v, sb0).wait()
    _lp_compute(ra0_v, rb0_v, sc_v)
    pltpu.sync_copy(sc_v, out_hbm.at[c, s, je])
    pltpu.make_async_copy(xf_hbm.at[aidx_v.at[je + 1]], ra1_v, sa1).wait()
    pltpu.make_async_copy(xf_hbm.at[bidx_v.at[je + 1]], rb1_v, sb1).wait()
    _lp_compute(ra1_v, rb1_v, sc_v)
    pltpu.sync_copy(sc_v, out_hbm.at[c, s, je + 1])


@functools.cache
def _lp_built():
    return pl.kernel(
        _lp_body,
        out_type=jax.ShapeDtypeStruct((NC, NS, NCHL, K, L), jnp.float32),
        mesh=_mesh(),
        scratch_types=[
            pltpu.VMEM((NCHL, K), jnp.int32),
            pltpu.VMEM((NCHL, K), jnp.int32),
            pltpu.VMEM((K, D), jnp.float32),
            pltpu.VMEM((K, D), jnp.float32),
            pltpu.VMEM((K, D), jnp.float32),
            pltpu.VMEM((K, D), jnp.float32),
            pltpu.VMEM((K, L), jnp.float32),
            pltpu.SemaphoreType.DMA,
            pltpu.SemaphoreType.DMA,
            pltpu.SemaphoreType.DMA,
            pltpu.SemaphoreType.DMA,
        ])


def _lp_kernel(xf_flat, a_pad, b_pad):
    return _lp_built()(xf_flat, a_pad, b_pad)


def _tc_d_body(p_ref, o_ref):
    o_ref[...] = jnp.sum(p_ref[...], axis=1)


def _elu(x):
    return jnp.where(x > 0, x, jnp.exp(x) - 1.0)


def _mm_t(a, w):
    return lax.dot_general(a, w, (((1,), (1,)), ((), ())),
                           preferred_element_type=jnp.float32)


def _att_rows(w_ref, bmat_ref, bb_ref):
    m_rows = []
    for i in range(V):
        t_i = lax.dot_general(w_ref[i], bmat_ref[...], (((1,), (0,)), ((), ())),
                              preferred_element_type=jnp.float32)
        for j in range(V):
            tot = jnp.sum(t_i * w_ref[j], axis=0, keepdims=True)
            tot = jnp.sum(tot, axis=1, keepdims=True)
            m_rows.append(jnp.broadcast_to(tot, (1, D)))
    rows = []
    for i in range(V):
        r = [m_rows[3 * i + j] + jnp.float32(D) * bb_ref[...] for j in range(V)]
        mx = jnp.maximum(jnp.maximum(r[0], r[1]), r[2])
        e = [jnp.exp(rj - mx) for rj in r]
        tot = e[0] + e[1] + e[2]
        rows.extend([ej / tot for ej in e])
    return jnp.concatenate(rows, axis=0)


def _tc_a_body(x_ref, hist_ref, w1_ref, b1m_ref, bb1_ref,
               xw1_ref, y1_ref, dis_ref, dinv_ref, att1_ref):
    g = pl.program_id(0)
    deg = jnp.sum(hist_ref[...], axis=(0, 3)) * (1.0 / L) + 1.0
    dis = lax.rsqrt(deg)
    dinv = 1.0 / deg
    dis_ref[...] = dis
    dinv_ref[...] = dinv
    xb = x_ref[...]
    for v in range(V):
        xw = _mm_t(xb, w1_ref[v])
        xw1_ref[v] = xw
        y1_ref[v] = xw * dis[v][:, None]

    @pl.when(g == 0)
    def _att():
        att1_ref[...] = _att_rows(w1_ref, b1m_ref, bb1_ref)


def _tc_b_body(part_ref, xw1_ref, dis_ref, dinv_ref, b1_ref, att_ref,
               wc_ref, bc_ref, w2_ref, b2m_ref, bb2_ref,
               xw2_ref, y2_ref, att2_ref):
    g = pl.program_id(0)
    dis = dis_ref[...]
    dinv = dinv_ref[...]
    xms = []
    for v in range(V):
        accv = part_ref[0, v] + part_ref[1, v]
        out1 = (accv * dis[v][:, None] + xw1_ref[v] * dinv[v][:, None]
                + b1_ref[v][None, :])
        temp = (att_ref[3 * v][None, :] * xw1_ref[0]
                + att_ref[3 * v + 1][None, :] * xw1_ref[1]
                + att_ref[3 * v + 2][None, :] * xw1_ref[2])
        xms.append((1.0 - ALPHA) + _elu(out1) + _elu(ALPHA * temp))
    xcat = jnp.concatenate(xms, axis=1)
    xc = _elu(_mm_t(xcat, wc_ref[...]) + bc_ref[...])
    for v in range(V):
        xw2 = _mm_t(xc, w2_ref[v])
        xw2_ref[v] = xw2
        y2_ref[v] = xw2 * dis[v][:, None]

    @pl.when(g == 0)
    def _att():
        att2_ref[...] = _att_rows(w2_ref, b2m_ref, bb2_ref)


def _tc_c_body(part_ref, xw2_ref, dis_ref, dinv_ref, b2_ref, att_ref, xf_ref):
    dis = dis_ref[...]
    dinv = dinv_ref[...]
    for v in range(V):
        accv = part_ref[0, v] + part_ref[1, v]
        out2 = (accv * dis[v][:, None] + xw2_ref[v] * dinv[v][:, None]
                + b2_ref[v][None, :])
        temp = (att_ref[3 * v][None, :] * xw2_ref[0]
                + att_ref[3 * v + 1][None, :] * xw2_ref[1]
                + att_ref[3 * v + 2][None, :] * xw2_ref[2])
        xf_ref[v] = (1.0 - ALPHA) + out2 + _elu(ALPHA * temp)


def _edges_pad(a, shift):
    a = a.reshape(V, NW, EPT)
    a = jnp.pad(a, ((0, 0), (0, 0), (0, NCH * K - EPT)), constant_values=PAD)
    if shift:
        a = a + (jnp.arange(V, dtype=jnp.int32) * NP)[:, None, None]
    return a.reshape(V, NC, NS, NCH, K)


def kernel(x, edge_index, edges, edges_neg, W1, b1, W2, b2,
           B1, bb1, B2, bb2, Wc, bc):
    f32 = jnp.float32
    x_p = jnp.pad(x, ((0, NP - N), (0, 0)))
    src_g = _edges_pad(edge_index[:, 0, :].astype(jnp.int32), True)
    dst_l = _edges_pad(edge_index[:, 1, :].astype(jnp.int32), False)
    bb1_row = jnp.broadcast_to(bb1.astype(f32).reshape(1, 1), (1, D))
    bb2_row = jnp.broadcast_to(bb2.astype(f32).reshape(1, 1), (1, D))

    hist = _deg_kernel(dst_l)

    full = lambda *dims: pl.BlockSpec(dims, lambda g: tuple(0 for _ in dims))
    xw1, y1, dis, dinv, att1 = pl.pallas_call(
        _tc_a_body,
        grid=(GRID,),
        in_specs=[
            pl.BlockSpec((BN, D), lambda g: (g, 0)),
            pl.BlockSpec((NC, V, BN, L), lambda g: (0, 0, g, 0)),
            full(V, D, D),
            full(D, D),
            full(1, D),
        ],
        out_specs=[
            pl.BlockSpec((V, BN, D), lambda g: (0, g, 0)),
            pl.BlockSpec((V, BN, D), lambda g: (0, g, 0)),
            pl.BlockSpec((V, BN), lambda g: (0, g)),
            pl.BlockSpec((V, BN), lambda g: (0, g)),
            full(9, D),
        ],
        out_shape=[
            jax.ShapeDtypeStruct((V, NP, D), f32),
            jax.ShapeDtypeStruct((V, NP, D), f32),
            jax.ShapeDtypeStruct((V, NP), f32),
            jax.ShapeDtypeStruct((V, NP), f32),
            jax.ShapeDtypeStruct((9, D), f32),
        ],
    )(x_p, hist, W1, B1, bb1_row)

    part1 = _scat_kernel(y1.reshape(V * NP, D), src_g, dst_l)

    xw2, y2, att2 = pl.pallas_call(
        _tc_b_body,
        grid=(GRID,),
        in_specs=[
            pl.BlockSpec((NC, V, BN, D), lambda g: (0, 0, g, 0)),
            pl.BlockSpec((V, BN, D), lambda g: (0, g, 0)),
            pl.BlockSpec((V, BN), lambda g: (0, g)),
            pl.BlockSpec((V, BN), lambda g: (0, g)),
            full(V, D),
            full(9, D),
            full(D, V * D),
            full(1, D),
            full(V, D, D),
            full(D, D),
            full(1, D),
        ],
        out_specs=[
            pl.BlockSpec((V, BN, D), lambda g: (0, g, 0)),
            pl.BlockSpec((V, BN, D), lambda g: (0, g, 0)),
            full(9, D),
        ],
        out_shape=[
            jax.ShapeDtypeStruct((V, NP, D), f32),
            jax.ShapeDtypeStruct((V, NP, D), f32),
            jax.ShapeDtypeStruct((9, D), f32),
        ],
    )(part1, xw1, dis, dinv, b1, att1, Wc, bc.reshape(1, D), W2, B2, bb2_row)

    part2 = _scat_kernel(y2.reshape(V * NP, D), src_g, dst_l)

    xf = pl.pallas_call(
        _tc_c_body,
        grid=(GRID,),
        in_specs=[
            pl.BlockSpec((NC, V, BN, D), lambda g: (0, 0, g, 0)),
            pl.BlockSpec((V, BN, D), lambda g: (0, g, 0)),
            pl.BlockSpec((V, BN), lambda g: (0, g)),
            pl.BlockSpec((V, BN), lambda g: (0, g)),
            full(V, D),
            full(9, D),
        ],
        out_specs=pl.BlockSpec((V, BN, D), lambda g: (0, g, 0)),
        out_shape=jax.ShapeDtypeStruct((V, NP, D), f32),
    )(part2, xw2, dis, dinv, b2, att2)

    shiftv = (jnp.arange(V, dtype=jnp.int32) * NP)[:, None]
    a_flat = (jnp.concatenate([edges[:, :, 0], edges_neg[:, :, 0]], axis=1)
              .astype(jnp.int32) + shiftv).reshape(TP)
    b_flat = (jnp.concatenate([edges[:, :, 1], edges_neg[:, :, 1]], axis=1)
              .astype(jnp.int32) + shiftv).reshape(TP)
    a_pad = jnp.pad(a_flat, (0, TPP - TP),
                    constant_values=PAD).reshape(NC, NS, NCHL, K)
    b_pad = jnp.pad(b_flat, (0, TPP - TP),
                    constant_values=PAD).reshape(NC, NS, NCHL, K)

    part_lp = _lp_kernel(xf.reshape(V * NP, D), a_pad, b_pad)

    BM = 8192
    flat = pl.pallas_call(
        _tc_d_body,
        grid=(TPP // BM,),
        in_specs=[pl.BlockSpec((BM, L), lambda i: (i, 0))],
        out_specs=pl.BlockSpec((BM,), lambda i: (i,)),
        out_shape=jax.ShapeDtypeStruct((TPP,), f32),
    )(part_lp.reshape(TPP, L))
    return flat[:TP].reshape(V, 2 * P)


# speedup vs baseline: 7.6189x; 1.0308x over previous
import functools

import jax
import jax.numpy as jnp
from jax import lax
from jax.experimental import pallas as pl
from jax.experimental.pallas import tpu as pltpu
from jax.experimental.pallas import tpu_sc as plsc

N = 10000
E = 320000
V = 3
D = 128
P = 50000
ALPHA = 0.5

NC = 2
NS = 16
L = 16
NW = NC * NS

NP = 10240
PAD = N
K = 128
EPT = E // NW
NCH = 80
HSZ = V * NP
ZR = 64
IB = 16

TP = 2 * V * P
NCHL = 74
PPT = NCHL * K
TPP = NW * PPT

BN = 1024
GRID = NP // BN
BF = 12500


DZR = 32


DW = 32


def _deg_body(dst_hbm, hist_hbm, didx_v, buf_v, zbuf_v, h16_v, acc_sh):
    c = lax.axis_index("c")
    s = lax.axis_index("s")
    zero = jnp.zeros((L,), jnp.float32)
    one = jnp.ones((L,), jnp.float32)

    @pl.loop(0, DZR)
    def _zfill(r):
        for t in range(DW // L):
            zbuf_v[r, pl.ds(t * L, L)] = zero

    for v in range(V):
        plsc.subcore_barrier()
        for q in range(640 // DZR):
            pltpu.sync_copy(zbuf_v, acc_sh.at[pl.ds(s * 640 + q * DZR, DZR)])

        @pl.loop(0, K)
        def _ofill(r):
            for t in range(DW // L):
                buf_v[r, pl.ds(t * L, L)] = one

        plsc.subcore_barrier()

        @pl.loop(0, NCH // IB)
        def _stage(st):
            pltpu.sync_copy(dst_hbm.at[v, c, s, pl.ds(st * IB, IB)], didx_v)

            @pl.loop(0, IB)
            def _chunks(j):
                pltpu.sync_copy(buf_v, acc_sh.at[didx_v.at[j]], add=True)

        plsc.subcore_barrier()
        for q in range(640 // K):
            pltpu.sync_copy(acc_sh.at[pl.ds(s * 640 + q * K, K)], buf_v)

            @pl.loop(0, K)
            def _cp(r):
                h16_v[r, :] = buf_v[r, pl.ds(0, L)]

            pltpu.sync_copy(h16_v,
                            hist_hbm.at[c, v, pl.ds(s * 640 + q * K, K)])


@functools.cache
def _mesh():
    return plsc.VectorSubcoreMesh(
        core_axis_name="c", subcore_axis_name="s",
        num_cores=NC, num_subcores=NS)


@functools.cache
def _deg_built():
    return pl.kernel(
        _deg_body,
        out_type=jax.ShapeDtypeStruct((NC, V, NP, L), jnp.float32),
        mesh=_mesh(),
        scratch_types=[
            pltpu.VMEM((IB, K), jnp.int32),
            pltpu.VMEM((K, DW), jnp.float32),
            pltpu.VMEM((DZR, DW), jnp.float32),
            pltpu.VMEM((K, L), jnp.float32),
            pltpu.VMEM_SHARED((NP, DW), jnp.float32),
        ])


def _deg_kernel(dst_l):
    return _deg_built()(dst_l)


def _scat_body(y_hbm, src_hbm, dst_hbm, out_hbm,
               sidx_v, didx_v, rows0_v, rows1_v, zrow_v, acc_sh,
               sem0, sem1, sem2, sem3):
    c = lax.axis_index("c")
    s = lax.axis_index("s")
    zero = jnp.zeros((L,), jnp.float32)

    @pl.loop(0, ZR)
    def _fill(r):
        for t in range(D // L):
            zrow_v[r, pl.ds(t * L, L)] = zero

    for v in range(V):
        plsc.subcore_barrier()
        for q in range(640 // ZR):
            pltpu.sync_copy(zrow_v, acc_sh.at[pl.ds(s * 640 + q * ZR, ZR)])
        plsc.subcore_barrier()

        @pl.loop(0, NCH // IB)
        def _stage(st):
            pltpu.sync_copy(src_hbm.at[v, c, s, pl.ds(st * IB, IB)], sidx_v)
            pltpu.sync_copy(dst_hbm.at[v, c, s, pl.ds(st * IB, IB)], didx_v)
            pltpu.async_copy(y_hbm.at[sidx_v.at[0]], rows0_v, sem0)
            pltpu.async_copy(y_hbm.at[sidx_v.at[1]], rows1_v, sem1)

            @pl.loop(0, (IB - 2) // 2)
            def _chunks(h):
                j = h * 2
                pltpu.make_async_copy(y_hbm.at[sidx_v.at[j]], rows0_v, sem0).wait()
                pltpu.async_copy(rows0_v, acc_sh.at[didx_v.at[j]], sem2, add=True)
                pltpu.make_async_copy(y_hbm.at[sidx_v.at[j + 1]], rows1_v, sem1).wait()
                pltpu.async_copy(rows1_v, acc_sh.at[didx_v.at[j + 1]], sem3, add=True)
                pltpu.make_async_copy(rows0_v, acc_sh.at[didx_v.at[j]], sem2).wait()
                pltpu.async_copy(y_hbm.at[sidx_v.at[j + 2]], rows0_v, sem0)
                pltpu.make_async_copy(rows1_v, acc_sh.at[didx_v.at[j + 1]], sem3).wait()
                pltpu.async_copy(y_hbm.at[sidx_v.at[j + 3]], rows1_v, sem1)

            je = IB - 2
            pltpu.make_async_copy(y_hbm.at[sidx_v.at[je]], rows0_v, sem0).wait()
            pltpu.async_copy(rows0_v, acc_sh.at[didx_v.at[je]], sem2, add=True)
            pltpu.make_async_copy(y_hbm.at[sidx_v.at[je + 1]], rows1_v, sem1).wait()
            pltpu.async_copy(rows1_v, acc_sh.at[didx_v.at[je + 1]], sem3, add=True)
            pltpu.make_async_copy(rows0_v, acc_sh.at[didx_v.at[je]], sem2).wait()
            pltpu.make_async_copy(rows1_v, acc_sh.at[didx_v.at[je + 1]], sem3).wait()

        plsc.subcore_barrier()
        for q in range(640 // K):
            pltpu.sync_copy(acc_sh.at[pl.ds(s * 640 + q * K, K)], rows0_v)
            pltpu.sync_copy(rows0_v, out_hbm.at[c, v, pl.ds(s * 640 + q * K, K)])


@functools.cache
def _scat_built():
    return pl.kernel(
        _scat_body,
        out_type=jax.ShapeDtypeStruct((NC, V, NP, D), jnp.float32),
        mesh=_mesh(),
        scratch_types=[
            pltpu.VMEM((IB, K), jnp.int32),
            pltpu.VMEM((IB, K), jnp.int32),
            pltpu.VMEM((K, D), jnp.float32),
            pltpu.VMEM((K, D), jnp.float32),
            pltpu.VMEM((ZR, D), jnp.float32),
            pltpu.VMEM_SHARED((NP, D), jnp.float32),
            pltpu.SemaphoreType.DMA,
            pltpu.SemaphoreType.DMA,
            pltpu.SemaphoreType.DMA,
            pltpu.SemaphoreType.DMA,
        ])


def _scat_kernel(y_flat, src_g, dst_l):
    return _scat_built()(y_flat, src_g, dst_l)


def _lp_compute(ra_v, rb_v, sc_v):
    @pl.loop(0, K)
    def _pairs(p):
        acc = ra_v[p, pl.ds(0, L)] * rb_v[p, pl.ds(0, L)]
        for t in range(1, D // L):
            acc = acc + ra_v[p, pl.ds(t * L, L)] * rb_v[p, pl.ds(t * L, L)]
        sc_v[p, :] = acc


def _lp_body(xf_hbm, aidx_hbm, bidx_hbm, out_hbm,
             aidx_v, bidx_v, ra0_v, rb0_v, ra1_v, rb1_v, sc_v,
             sa0, sb0, sa1, sb1):
    c = lax.axis_index("c")
    s = lax.axis_index("s")
    pltpu.sync_copy(aidx_hbm.at[c, s], aidx_v)
    pltpu.sync_copy(bidx_hbm.at[c, s], bidx_v)

    pltpu.async_copy(xf_hbm.at[aidx_v.at[0]], ra0_v, sa0)
    pltpu.async_copy(xf_hbm.at[bidx_v.at[0]], rb0_v, sb0)
    pltpu.async_copy(xf_hbm.at[aidx_v.at[1]], ra1_v, sa1)
    pltpu.async_copy(xf_hbm.at[bidx_v.at[1]], rb1_v, sb1)

    @pl.loop(0, (NCHL - 2) // 2)
    def _chunks(h):
        j = h * 2
        pltpu.make_async_copy(xf_hbm.at[aidx_v.at[j]], ra0_v, sa0).wait()
        pltpu.make_async_copy(xf_hbm.at[bidx_v.at[j]], rb0_v, sb0).wait()
        _lp_compute(ra0_v, rb0_v, sc_v)
        pltpu.sync_copy(sc_v, out_hbm.at[c, s, j])
        pltpu.async_copy(xf_hbm.at[aidx_v.at[j + 2]], ra0_v, sa0)
        pltpu.async_copy(xf_hbm.at[bidx_v.at[j + 2]], rb0_v, sb0)
        pltpu.make_async_copy(xf_hbm.at[aidx_v.at[j + 1]], ra1_v, sa1).wait()
        pltpu.make_async_copy(xf_hbm.at[bidx_v.at[j + 1]], rb1_v, sb1).wait()
        _lp_compute(ra1_v, rb1_v, sc_v)
        pltpu.sync_copy(sc_v, out_hbm.at[c, s, j + 1])
        pltpu.async_copy(xf_hbm.at[aidx_v.at[j + 3]], ra1_v, sa1)
        pltpu.async_copy(xf_hbm.at[bidx_v.at[j + 3]], rb1_v, sb1)

    je = NCHL - 2
    pltpu.make_async_copy(xf_hbm.at[aidx_v.at[je]], ra0_v, sa0).wait()
    pltpu.make_async_copy(xf_hbm.at[bidx_v.at[je]], rb0_v, sb0).wait()
    _lp_compute(ra0_v, rb0_v, sc_v)
    pltpu.sync_copy(sc_v, out_hbm.at[c, s, je])
    pltpu.make_async_copy(xf_hbm.at[aidx_v.at[je + 1]], ra1_v, sa1).wait()
    pltpu.make_async_copy(xf_hbm.at[bidx_v.at[je + 1]], rb1_v, sb1).wait()
    _lp_compute(ra1_v, rb1_v, sc_v)
    pltpu.sync_copy(sc_v, out_hbm.at[c, s, je + 1])


@functools.cache
def _lp_built():
    return pl.kernel(
        _lp_body,
        out_type=jax.ShapeDtypeStruct((NC, NS, NCHL, K, L), jnp.float32),
        mesh=_mesh(),
        scratch_types=[
            pltpu.VMEM((NCHL, K), jnp.int32),
            pltpu.VMEM((NCHL, K), jnp.int32),
            pltpu.VMEM((K, D), jnp.float32),
            pltpu.VMEM((K, D), jnp.float32),
            pltpu.VMEM((K, D), jnp.float32),
            pltpu.VMEM((K, D), jnp.float32),
            pltpu.VMEM((K, L), jnp.float32),
            pltpu.SemaphoreType.DMA,
            pltpu.SemaphoreType.DMA,
            pltpu.SemaphoreType.DMA,
            pltpu.SemaphoreType.DMA,
        ])


def _lp_kernel(xf_flat, a_pad, b_pad):
    return _lp_built()(xf_flat, a_pad, b_pad)


def _tc_d_body(p_ref, o_ref):
    o_ref[...] = jnp.sum(p_ref[...], axis=1)


def _elu(x):
    return jnp.where(x > 0, x, jnp.exp(x) - 1.0)


def _mm_t(a, w):
    return lax.dot_general(a, w, (((1,), (1,)), ((), ())),
                           preferred_element_type=jnp.float32)


def _att_rows(w_ref, bmat_ref, bb_ref):
    m_rows = []
    for i in range(V):
        t_i = lax.dot_general(w_ref[i], bmat_ref[...], (((1,), (0,)), ((), ())),
                              preferred_element_type=jnp.float32)
        for j in range(V):
            tot = jnp.sum(t_i * w_ref[j], axis=0, keepdims=True)
            tot = jnp.sum(tot, axis=1, keepdims=True)
            m_rows.append(jnp.broadcast_to(tot, (1, D)))
    rows = []
    for i in range(V):
        r = [m_rows[3 * i + j] + jnp.float32(D) * bb_ref[...] for j in range(V)]
        mx = jnp.maximum(jnp.maximum(r[0], r[1]), r[2])
        e = [jnp.exp(rj - mx) for rj in r]
        tot = e[0] + e[1] + e[2]
        rows.extend([ej / tot for ej in e])
    return jnp.concatenate(rows, axis=0)


def _tc_a_body(x_ref, hist_ref, w1_ref, b1m_ref, bb1_ref,
               xw1_ref, y1_ref, dis_ref, dinv_ref, att1_ref):
    g = pl.program_id(0)
    deg = jnp.sum(hist_ref[...], axis=(0, 3)) * (1.0 / L) + 1.0
    dis = lax.rsqrt(deg)
    dinv = 1.0 / deg
    dis_ref[...] = dis
    dinv_ref[...] = dinv
    xb = x_ref[...]
    for v in range(V):
        xw = _mm_t(xb, w1_ref[v])
        xw1_ref[v] = xw
        y1_ref[v] = xw * dis[v][:, None]

    @pl.when(g == 0)
    def _att():
        att1_ref[...] = _att_rows(w1_ref, b1m_ref, bb1_ref)


def _tc_b_body(part_ref, xw1_ref, dis_ref, dinv_ref, b1_ref, att_ref,
               wc_ref, bc_ref, w2_ref, b2m_ref, bb2_ref,
               xw2_ref, y2_ref, att2_ref):
    g = pl.program_id(0)
    dis = dis_ref[...]
    dinv = dinv_ref[...]
    xms = []
    for v in range(V):
        accv = part_ref[0, v] + part_ref[1, v]
        out1 = (accv * dis[v][:, None] + xw1_ref[v] * dinv[v][:, None]
                + b1_ref[v][None, :])
        temp = (att_ref[3 * v][None, :] * xw1_ref[0]
                + att_ref[3 * v + 1][None, :] * xw1_ref[1]
                + att_ref[3 * v + 2][None, :] * xw1_ref[2])
        xms.append((1.0 - ALPHA) + _elu(out1) + _elu(ALPHA * temp))
    xcat = jnp.concatenate(xms, axis=1)
    xc = _elu(_mm_t(xcat, wc_ref[...]) + bc_ref[...])
    for v in range(V):
        xw2 = _mm_t(xc, w2_ref[v])
        xw2_ref[v] = xw2
        y2_ref[v] = xw2 * dis[v][:, None]

    @pl.when(g == 0)
    def _att():
        att2_ref[...] = _att_rows(w2_ref, b2m_ref, bb2_ref)


def _tc_c_body(part_ref, xw2_ref, dis_ref, dinv_ref, b2_ref, att_ref, xf_ref):
    dis = dis_ref[...]
    dinv = dinv_ref[...]
    for v in range(V):
        accv = part_ref[0, v] + part_ref[1, v]
        out2 = (accv * dis[v][:, None] + xw2_ref[v] * dinv[v][:, None]
                + b2_ref[v][None, :])
        temp = (att_ref[3 * v][None, :] * xw2_ref[0]
                + att_ref[3 * v + 1][None, :] * xw2_ref[1]
                + att_ref[3 * v + 2][None, :] * xw2_ref[2])
        xf_ref[v] = (1.0 - ALPHA) + out2 + _elu(ALPHA * temp)


def _edges_pad(a, shift):
    a = a.reshape(V, NW, EPT)
    a = jnp.pad(a, ((0, 0), (0, 0), (0, NCH * K - EPT)), constant_values=PAD)
    if shift:
        a = a + (jnp.arange(V, dtype=jnp.int32) * NP)[:, None, None]
    return a.reshape(V, NC, NS, NCH, K)


def kernel(x, edge_index, edges, edges_neg, W1, b1, W2, b2,
           B1, bb1, B2, bb2, Wc, bc):
    f32 = jnp.float32
    x_p = jnp.pad(x, ((0, NP - N), (0, 0)))
    src_g = _edges_pad(edge_index[:, 0, :].astype(jnp.int32), True)
    dst_l = _edges_pad(edge_index[:, 1, :].astype(jnp.int32), False)
    bb1_row = jnp.broadcast_to(bb1.astype(f32).reshape(1, 1), (1, D))
    bb2_row = jnp.broadcast_to(bb2.astype(f32).reshape(1, 1), (1, D))

    hist = _deg_kernel(dst_l)

    full = lambda *dims: pl.BlockSpec(dims, lambda g: tuple(0 for _ in dims))
    xw1, y1, dis, dinv, att1 = pl.pallas_call(
        _tc_a_body,
        grid=(GRID,),
        in_specs=[
            pl.BlockSpec((BN, D), lambda g: (g, 0)),
            pl.BlockSpec((NC, V, BN, L), lambda g: (0, 0, g, 0)),
            full(V, D, D),
            full(D, D),
            full(1, D),
        ],
        out_specs=[
            pl.BlockSpec((V, BN, D), lambda g: (0, g, 0)),
            pl.BlockSpec((V, BN, D), lambda g: (0, g, 0)),
            pl.BlockSpec((V, BN), lambda g: (0, g)),
            pl.BlockSpec((V, BN), lambda g: (0, g)),
            full(9, D),
        ],
        out_shape=[
            jax.ShapeDtypeStruct((V, NP, D), f32),
            jax.ShapeDtypeStruct((V, NP, D), f32),
            jax.ShapeDtypeStruct((V, NP), f32),
            jax.ShapeDtypeStruct((V, NP), f32),
            jax.ShapeDtypeStruct((9, D), f32),
        ],
    )(x_p, hist, W1, B1, bb1_row)

    part1 = _scat_kernel(y1.reshape(V * NP, D), src_g, dst_l)

    xw2, y2, att2 = pl.pallas_call(
        _tc_b_body,
        grid=(GRID,),
        in_specs=[
            pl.BlockSpec((NC, V, BN, D), lambda g: (0, 0, g, 0)),
            pl.BlockSpec((V, BN, D), lambda g: (0, g, 0)),
            pl.BlockSpec((V, BN), lambda g: (0, g)),
            pl.BlockSpec((V, BN), lambda g: (0, g)),
            full(V, D),
            full(9, D),
            full(D, V * D),
            full(1, D),
            full(V, D, D),
            full(D, D),
            full(1, D),
        ],
        out_specs=[
            pl.BlockSpec((V, BN, D), lambda g: (0, g, 0)),
            pl.BlockSpec((V, BN, D), lambda g: (0, g, 0)),
            full(9, D),
        ],
        out_shape=[
            jax.ShapeDtypeStruct((V, NP, D), f32),
            jax.ShapeDtypeStruct((V, NP, D), f32),
            jax.ShapeDtypeStruct((9, D), f32),
        ],
    )(part1, xw1, dis, dinv, b1, att1, Wc, bc.reshape(1, D), W2, B2, bb2_row)

    part2 = _scat_kernel(y2.reshape(V * NP, D), src_g, dst_l)

    xf = pl.pallas_call(
        _tc_c_body,
        grid=(GRID,),
        in_specs=[
            pl.BlockSpec((NC, V, BN, D), lambda g: (0, 0, g, 0)),
            pl.BlockSpec((V, BN, D), lambda g: (0, g, 0)),
            pl.BlockSpec((V, BN), lambda g: (0, g)),
            pl.BlockSpec((V, BN), lambda g: (0, g)),
            full(V, D),
            full(9, D),
        ],
        out_specs=pl.BlockSpec((V, BN, D), lambda g: (0, g, 0)),
        out_shape=jax.ShapeDtypeStruct((V, NP, D), f32),
    )(part2, xw2, dis, dinv, b2, att2)

    shiftv = (jnp.arange(V, dtype=jnp.int32) * NP)[:, None]
    a_flat = (jnp.concatenate([edges[:, :, 0], edges_neg[:, :, 0]], axis=1)
              .astype(jnp.int32) + shiftv).reshape(TP)
    b_flat = (jnp.concatenate([edges[:, :, 1], edges_neg[:, :, 1]], axis=1)
              .astype(jnp.int32) + shiftv).reshape(TP)
    a_pad = jnp.pad(a_flat, (0, TPP - TP),
                    constant_values=PAD).reshape(NC, NS, NCHL, K)
    b_pad = jnp.pad(b_flat, (0, TPP - TP),
                    constant_values=PAD).reshape(NC, NS, NCHL, K)

    part_lp = _lp_kernel(xf.reshape(V * NP, D), a_pad, b_pad)

    BM = 8192
    flat = pl.pallas_call(
        _tc_d_body,
        grid=(TPP // BM,),
        in_specs=[pl.BlockSpec((BM, L), lambda i: (i, 0))],
        out_specs=pl.BlockSpec((BM,), lambda i: (i,)),
        out_shape=jax.ShapeDtypeStruct((TPP,), f32),
    )(part_lp.reshape(TPP, L))
    return flat[:TP].reshape(V, 2 * P)
